# baseline (speedup 1.0000x reference)
import jax
import jax.numpy as jnp
from jax.experimental import pallas as pl
from jax.experimental.pallas import tpu as pltpu

N = 10000
E = 640000
ATOM = 128
H = 128
HEADS = 4
DH = 32
G = 256
T = 5


def _heads_body(hh_ref, w1_ref, b1_ref, w2_ref, b2_ref, out_ref):
    hh = hh_ref[...]
    for t in range(T):
        z = jnp.maximum(jnp.dot(hh, w1_ref[t], preferred_element_type=jnp.float32) + b1_ref[t], 0.0)
        out_ref[:, t] = (jnp.dot(z, w2_ref[t], preferred_element_type=jnp.float32) + b2_ref[t])[:, 0]


def _heads(hh, w1, b1, w2, b2):
    return pl.pallas_call(
        _heads_body,
        out_shape=jax.ShapeDtypeStruct((G, T), jnp.float32),
    )(hh, w1, b1, w2, b2)


def kernel(x, params, edge_index, batch):
    p = params
    src, dst = edge_index[0], edge_index[1]
    loop = jnp.arange(N)
    srcl = jnp.concatenate([src, loop])
    dstl = jnp.concatenate([dst, loop])
    deg = jax.ops.segment_sum(jnp.ones_like(dstl, dtype=x.dtype), dstl, num_segments=N)
    dinv = jax.lax.rsqrt(jnp.maximum(deg, 1.0))
    norm = dinv[srcl] * dinv[dstl]
    h = x
    for i in range(3):
        hw = h @ p['W%d' % i]
        out = jax.ops.segment_sum(hw[srcl] * norm[:, None], dstl, num_segments=N) + p['b%d' % i]
        m = out.mean(axis=0)
        v = out.var(axis=0)
        h = jnp.maximum((out - m) / jnp.sqrt(v + 1e-5) * p['bn_g%d' % i] + p['bn_b%d' % i], 0.0)
    hg = (h @ p['Wg']).reshape(N, HEADS, DH)
    as_ = (hg * p['att_src'][None]).sum(-1)
    ad_ = (hg * p['att_dst'][None]).sum(-1)
    e = jax.nn.leaky_relu(as_[srcl] + ad_[dstl], 0.2)
    emax = jax.ops.segment_max(e, dstl, num_segments=N)
    emax = jnp.where(jnp.isfinite(emax), emax, 0.0)
    ex = jnp.exp(e - emax[dstl])
    den = jax.ops.segment_sum(ex, dstl, num_segments=N)
    alpha = ex / jnp.maximum(den[dstl], 1e-16)
    gat = jax.ops.segment_sum(alpha[:, :, None] * hg[srcl], dstl, num_segments=N)
    gat = gat.reshape(N, HEADS * DH) + p['bg']
    cnt = jax.ops.segment_sum(jnp.ones((N,), gat.dtype), batch, num_segments=G)
    mean_pool = jax.ops.segment_sum(gat, batch, num_segments=G) / jnp.maximum(cnt, 1.0)[:, None]
    mx = jax.ops.segment_max(gat, batch, num_segments=G)
    max_pool = jnp.where(cnt[:, None] > 0, mx, 0.0)
    hh = jnp.concatenate([mean_pool, max_pool], axis=1)
    w1 = jnp.stack([p['h%d_W1' % t] for t in range(T)])
    b1 = jnp.stack([p['h%d_b1' % t] for t in range(T)])
    w2 = jnp.stack([p['h%d_W2' % t] for t in range(T)])
    b2 = jnp.stack([p['h%d_b2' % t] for t in range(T)])
    return _heads(hh, w1, b1, w2, b2)


# speedup vs baseline: 1.2493x; 1.2493x over previous
import functools
import jax
import jax.numpy as jnp
from jax import lax
from jax.experimental import pallas as pl
from jax.experimental.pallas import tpu as pltpu
from jax.experimental.pallas import tpu_sc as plsc

N = 10000
E = 640000
ATOM = 128
H = 128
HEADS = 4
DH = 32
G = 256
T = 5

NC = 2
NS = 16
NW = NC * NS
CHUNK = 128
CPW = 160
EPW = CHUNK * CPW
EPAD = EPW * NW
NBLK = EPAD // CHUNK
NPAD = 10240
RPT = NPAD // NS
DPB = NBLK // NS
DNW = NPAD // NW
SEG = 4
CPS = CPW // SEG

_mesh = plsc.VectorSubcoreMesh(
    core_axis_name="c", subcore_axis_name="s", num_cores=NC, num_subcores=NS)


def _vslices(n):
    return range(n // 16)


def _deg_dinv(dst2d):
    @functools.partial(
        pl.kernel,
        out_type=jax.ShapeDtypeStruct((NPAD,), jnp.float32),
        mesh=_mesh,
        compiler_params=pltpu.CompilerParams(use_tc_tiling_on_sc=False, needs_layout_passes=False),
        scratch_types=[
            pltpu.VMEM((DPB, 128), jnp.int32),
            pltpu.VMEM((NPAD,), jnp.float32),
            pltpu.VMEM((DNW,), jnp.float32),
            pltpu.VMEM((DNW,), jnp.float32),
            pltpu.VMEM_SHARED((NS, NPAD), jnp.float32),
        ],
    )
    def k(dst_hbm, out_hbm, idx_v, pdeg, buf, tmp, sdeg):
        c = lax.axis_index("c")
        s = lax.axis_index("s")
        w = s * NC + c
        pltpu.sync_copy(dst_hbm.at[pl.ds(s * DPB, DPB)], idx_v)

        def z(i, carry):
            pdeg[pl.ds(i * 16, 16)] = jnp.zeros((16,), jnp.float32)
            return carry
        lax.fori_loop(0, NPAD // 16, z, 0)

        ones = jnp.ones((16,), jnp.float32)

        def cnt(j, carry):
            for g in range(8):
                idx = idx_v[j, pl.ds(g * 16, 16)]
                plsc.addupdate_scatter(pdeg, [idx], ones)
            return carry
        lax.fori_loop(0, DPB, cnt, 0)

        pltpu.sync_copy(pdeg, sdeg.at[s])
        plsc.subcore_barrier()

        base = w * DNW

        def zz(i, carry):
            buf[pl.ds(i * 16, 16)] = jnp.zeros((16,), jnp.float32)
            return carry
        lax.fori_loop(0, DNW // 16, zz, 0)

        def mrg(t, carry):
            pltpu.sync_copy(sdeg.at[t, pl.ds(base, DNW)], tmp)

            def add(i, c2):
                sl = pl.ds(i * 16, 16)
                buf[sl] = buf[sl] + tmp[sl]
                return c2
            lax.fori_loop(0, DNW // 16, add, 0)
            return carry
        lax.fori_loop(0, NS, mrg, 0)

        def rs(i, carry):
            sl = pl.ds(i * 16, 16)
            d = buf[sl] + 1.0
            ii = plsc.bitcast(d, jnp.int32)
            ii = 0x5F3759DF - jax.lax.shift_right_logical(ii, 1)
            y = plsc.bitcast(ii, jnp.float32)
            for _ in range(3):
                y = y * (1.5 - 0.5 * d * y * y)
            buf[sl] = y
            return carry
        lax.fori_loop(0, DNW // 16, rs, 0)

        pltpu.sync_copy(buf, out_hbm.at[pl.ds(base, DNW)])

    return k(dst2d)


def _spmm(table, src2d, dst2d, z128):
    @functools.partial(
        pl.kernel,
        out_type=jax.ShapeDtypeStruct((NC, NPAD, H), jnp.float32),
        mesh=_mesh,
        compiler_params=pltpu.CompilerParams(use_tc_tiling_on_sc=False, needs_layout_passes=False),
        scratch_types=[
            pltpu.VMEM((CPS, 128), jnp.int32),
            pltpu.VMEM((CPS, 128), jnp.int32),
            pltpu.VMEM_SHARED((NPAD, H), jnp.float32),
            pltpu.VMEM((CHUNK, H), jnp.float32),
            pltpu.VMEM((CHUNK, H), jnp.float32),
            pltpu.SemaphoreType.DMA,
            pltpu.SemaphoreType.DMA,
        ],
    )
    def k(tab, srcb, dstb, zb, out, src_v, dst_v, acc, r0, r1, s0, s1):
        c = lax.axis_index("c")
        s = lax.axis_index("s")
        w = s * NC + c
        pltpu.sync_copy(zb.at[pl.ds(s * RPT, RPT)], acc.at[pl.ds(s * RPT, RPT)])
        plsc.subcore_barrier()

        def seg(sg, carry):
            base = w * CPW + sg * CPS
            pltpu.sync_copy(srcb.at[pl.ds(base, CPS)], src_v)
            pltpu.sync_copy(dstb.at[pl.ds(base, CPS)], dst_v)

            def it(jp, c2):
                j = jp * 2
                d0 = pltpu.async_copy(tab.at[src_v.at[j]], r0, s0)
                d1 = pltpu.async_copy(tab.at[src_v.at[j + 1]], r1, s1)
                d0.wait()
                pltpu.sync_copy(r0, acc.at[dst_v.at[j]], add=True)
                d1.wait()
                pltpu.sync_copy(r1, acc.at[dst_v.at[j + 1]], add=True)
                return c2
            lax.fori_loop(0, CPS // 2, it, 0)
            return carry
        lax.fori_loop(0, SEG, seg, 0)

        plsc.subcore_barrier()
        pltpu.sync_copy(acc.at[pl.ds(s * RPT, RPT)],
                        out.at[c, pl.ds(s * RPT, RPT)])

    return k(table, src2d, dst2d, z128)


def _tc_first(x, w0, dinv_col):
    def body(x_ref, w_ref, di_ref, o_ref):
        o_ref[...] = jnp.dot(x_ref[...], w_ref[...],
                             preferred_element_type=jnp.float32) * di_ref[...]
    return pl.pallas_call(
        body, out_shape=jax.ShapeDtypeStruct((N, H), jnp.float32))(
            x, w0, dinv_col)


def _bn_relu(u, g, bb):
    m = jnp.mean(u, axis=0, keepdims=True)
    d = u - m
    v = jnp.mean(d * d, axis=0, keepdims=True)
    return jnp.maximum(d * jax.lax.rsqrt(v + 1e-5) * g + bb, 0.0)


def _tc_mid(P, hs, dinv_col, b, g, bb, wn):
    def body(p_ref, hs_ref, di_ref, b_ref, g_ref, bb_ref, w_ref, o_ref):
        u = (p_ref[0, :N, :] + p_ref[1, :N, :] + hs_ref[...]) * di_ref[...] \
            + b_ref[...]
        hr = _bn_relu(u, g_ref[...], bb_ref[...])
        o_ref[...] = jnp.dot(hr, w_ref[...],
                             preferred_element_type=jnp.float32) * di_ref[...]
    return pl.pallas_call(
        body, out_shape=jax.ShapeDtypeStruct((N, H), jnp.float32))(
            P, hs, dinv_col, b, g, bb, wn)


def _tc_last(P, hs, dinv_col, b, g, bb):
    def body(p_ref, hs_ref, di_ref, b_ref, g_ref, bb_ref, o_ref):
        u = (p_ref[0, :N, :] + p_ref[1, :N, :] + hs_ref[...]) * di_ref[...] \
            + b_ref[...]
        o_ref[...] = _bn_relu(u, g_ref[...], bb_ref[...])
    return pl.pallas_call(
        body, out_shape=jax.ShapeDtypeStruct((N, H), jnp.float32))(
            P, hs, dinv_col, b, g, bb)


def _heads_body(hh_ref, w1_ref, b1_ref, w2_ref, b2_ref, out_ref):
    hh = hh_ref[...]
    for t in range(T):
        z = jnp.maximum(
            jnp.dot(hh, w1_ref[t], preferred_element_type=jnp.float32)
            + b1_ref[t], 0.0)
        out_ref[:, t] = (jnp.dot(z, w2_ref[t],
                                 preferred_element_type=jnp.float32)
                         + b2_ref[t])[:, 0]


def _heads(hh, w1, b1, w2, b2):
    return pl.pallas_call(
        _heads_body,
        out_shape=jax.ShapeDtypeStruct((G, T), jnp.float32),
    )(hh, w1, b1, w2, b2)


def kernel(x, params, edge_index, batch):
    p = params
    src = edge_index[0].astype(jnp.int32)
    dst = edge_index[1].astype(jnp.int32)
    pad = EPAD - E
    src2d = jnp.concatenate(
        [src, jnp.zeros((pad,), jnp.int32)]).reshape(NBLK, 128)
    dst2d = jnp.concatenate(
        [dst, jnp.full((pad,), N, jnp.int32)]).reshape(NBLK, 128)
    z128 = jnp.zeros((NPAD, H), jnp.float32)

    dinv = _deg_dinv(dst2d)
    dinv_col = dinv[:N, None]

    h1s = _tc_first(x, p['W0'], dinv_col)
    P1 = _spmm(h1s, src2d, dst2d, z128)
    h2s = _tc_mid(P1, h1s, dinv_col, p['b0'], p['bn_g0'], p['bn_b0'], p['W1'])
    P2 = _spmm(h2s, src2d, dst2d, z128)
    h3s = _tc_mid(P2, h2s, dinv_col, p['b1'], p['bn_g1'], p['bn_b1'], p['W2'])
    P3 = _spmm(h3s, src2d, dst2d, z128)
    h = _tc_last(P3, h3s, dinv_col, p['b2'], p['bn_g2'], p['bn_b2'])

    loop = jnp.arange(N)
    srcl = jnp.concatenate([src, loop])
    dstl = jnp.concatenate([dst, loop])
    hg = (h @ p['Wg']).reshape(N, HEADS, DH)
    as_ = (hg * p['att_src'][None]).sum(-1)
    ad_ = (hg * p['att_dst'][None]).sum(-1)
    e = jax.nn.leaky_relu(as_[srcl] + ad_[dstl], 0.2)
    emax = jax.ops.segment_max(e, dstl, num_segments=N)
    emax = jnp.where(jnp.isfinite(emax), emax, 0.0)
    ex = jnp.exp(e - emax[dstl])
    den = jax.ops.segment_sum(ex, dstl, num_segments=N)
    alpha = ex / jnp.maximum(den[dstl], 1e-16)
    gat = jax.ops.segment_sum(alpha[:, :, None] * hg[srcl], dstl,
                              num_segments=N)
    gat = gat.reshape(N, HEADS * DH) + p['bg']
    cnt = jax.ops.segment_sum(jnp.ones((N,), gat.dtype), batch,
                              num_segments=G)
    mean_pool = jax.ops.segment_sum(gat, batch, num_segments=G) \
        / jnp.maximum(cnt, 1.0)[:, None]
    mx = jax.ops.segment_max(gat, batch, num_segments=G)
    max_pool = jnp.where(cnt[:, None] > 0, mx, 0.0)
    hh = jnp.concatenate([mean_pool, max_pool], axis=1)
    w1 = jnp.stack([p['h%d_W1' % t] for t in range(T)])
    b1 = jnp.stack([p['h%d_b1' % t] for t in range(T)])
    w2 = jnp.stack([p['h%d_W2' % t] for t in range(T)])
    b2 = jnp.stack([p['h%d_b2' % t] for t in range(T)])
    return _heads(hh, w1, b1, w2, b2)


# speedup vs baseline: 17.3390x; 13.8786x over previous
import functools
import jax
import jax.numpy as jnp
from jax import lax
from jax.experimental import pallas as pl
from jax.experimental.pallas import tpu as pltpu
from jax.experimental.pallas import tpu_sc as plsc

N = 10000
E = 640000
ATOM = 128
H = 128
HEADS = 4
DH = 32
G = 256
T = 5

NC = 2
NS = 16
NW = NC * NS
CHUNK = 128
CPW = 160
EPW = CHUNK * CPW
EPAD = EPW * NW
NBLK = EPAD // CHUNK
NPAD = 10240
RPT = NPAD // NS
DPB = NBLK // NS
DNW = NPAD // NW
SEG = 4
CPS = CPW // SEG
GSEG = 8
GCPS = CPW // GSEG
NP2 = 10016
NQ = 4
NPQ = NP2 // NQ
GP = 272

_mesh = plsc.VectorSubcoreMesh(
    core_axis_name="c", subcore_axis_name="s", num_cores=NC, num_subcores=NS)


def _vslices(n):
    return range(n // 16)


def _deg_dinv(dst2d):
    @functools.partial(
        pl.kernel,
        out_type=jax.ShapeDtypeStruct((NPAD,), jnp.float32),
        mesh=_mesh,
        compiler_params=pltpu.CompilerParams(use_tc_tiling_on_sc=False, needs_layout_passes=False),
        scratch_types=[
            pltpu.VMEM((DPB, 128), jnp.int32),
            pltpu.VMEM((NPAD,), jnp.float32),
            pltpu.VMEM((DNW,), jnp.float32),
            pltpu.VMEM((DNW,), jnp.float32),
            pltpu.VMEM_SHARED((NS, NPAD), jnp.float32),
        ],
    )
    def k(dst_hbm, out_hbm, idx_v, pdeg, buf, tmp, sdeg):
        c = lax.axis_index("c")
        s = lax.axis_index("s")
        w = s * NC + c
        pltpu.sync_copy(dst_hbm.at[pl.ds(s * DPB, DPB)], idx_v)

        def z(i, carry):
            pdeg[pl.ds(i * 16, 16)] = jnp.zeros((16,), jnp.float32)
            return carry
        lax.fori_loop(0, NPAD // 16, z, 0)

        ones = jnp.ones((16,), jnp.float32)

        def cnt(j, carry):
            for g in range(8):
                idx = idx_v[j, pl.ds(g * 16, 16)]
                plsc.addupdate_scatter(pdeg, [idx], ones)
            return carry
        lax.fori_loop(0, DPB, cnt, 0)

        pltpu.sync_copy(pdeg, sdeg.at[s])
        plsc.subcore_barrier()

        base = w * DNW

        def zz(i, carry):
            buf[pl.ds(i * 16, 16)] = jnp.zeros((16,), jnp.float32)
            return carry
        lax.fori_loop(0, DNW // 16, zz, 0)

        def mrg(t, carry):
            pltpu.sync_copy(sdeg.at[t, pl.ds(base, DNW)], tmp)

            def add(i, c2):
                sl = pl.ds(i * 16, 16)
                buf[sl] = buf[sl] + tmp[sl]
                return c2
            lax.fori_loop(0, DNW // 16, add, 0)
            return carry
        lax.fori_loop(0, NS, mrg, 0)

        def rs(i, carry):
            sl = pl.ds(i * 16, 16)
            d = buf[sl] + 1.0
            ii = plsc.bitcast(d, jnp.int32)
            ii = 0x5F3759DF - jax.lax.shift_right_logical(ii, 1)
            y = plsc.bitcast(ii, jnp.float32)
            for _ in range(3):
                y = y * (1.5 - 0.5 * d * y * y)
            buf[sl] = y
            return carry
        lax.fori_loop(0, DNW // 16, rs, 0)

        pltpu.sync_copy(buf, out_hbm.at[pl.ds(base, DNW)])

    return k(dst2d)


def _spmm(table, src2d, dst2d, z128):
    @functools.partial(
        pl.kernel,
        out_type=jax.ShapeDtypeStruct((NC, NPAD, H), jnp.float32),
        mesh=_mesh,
        compiler_params=pltpu.CompilerParams(use_tc_tiling_on_sc=False, needs_layout_passes=False),
        scratch_types=[
            pltpu.VMEM((CPS, 128), jnp.int32),
            pltpu.VMEM((CPS, 128), jnp.int32),
            pltpu.VMEM_SHARED((NPAD, H), jnp.float32),
            pltpu.VMEM((CHUNK, H), jnp.float32),
            pltpu.VMEM((CHUNK, H), jnp.float32),
            pltpu.SemaphoreType.DMA,
            pltpu.SemaphoreType.DMA,
        ],
    )
    def k(tab, srcb, dstb, zb, out, src_v, dst_v, acc, r0, r1, s0, s1):
        c = lax.axis_index("c")
        s = lax.axis_index("s")
        w = s * NC + c
        pltpu.sync_copy(zb.at[pl.ds(s * RPT, RPT)], acc.at[pl.ds(s * RPT, RPT)])
        plsc.subcore_barrier()

        def seg(sg, carry):
            base = w * CPW + sg * CPS
            pltpu.sync_copy(srcb.at[pl.ds(base, CPS)], src_v)
            pltpu.sync_copy(dstb.at[pl.ds(base, CPS)], dst_v)

            def it(jp, c2):
                j = jp * 2
                d0 = pltpu.async_copy(tab.at[src_v.at[j]], r0, s0)
                d1 = pltpu.async_copy(tab.at[src_v.at[j + 1]], r1, s1)
                d0.wait()
                pltpu.sync_copy(r0, acc.at[dst_v.at[j]], add=True)
                d1.wait()
                pltpu.sync_copy(r1, acc.at[dst_v.at[j + 1]], add=True)
                return c2
            lax.fori_loop(0, CPS // 2, it, 0)
            return carry
        lax.fori_loop(0, SEG, seg, 0)

        plsc.subcore_barrier()
        pltpu.sync_copy(acc.at[pl.ds(s * RPT, RPT)],
                        out.at[c, pl.ds(s * RPT, RPT)])

    return k(table, src2d, dst2d, z128)


def _gat_sc(hgs, adc, src2d, dst2d, z144):
    @functools.partial(
        pl.kernel,
        out_type=jax.ShapeDtypeStruct((NC, NPAD, 144), jnp.float32),
        mesh=_mesh,
        compiler_params=pltpu.CompilerParams(
            use_tc_tiling_on_sc=False, needs_layout_passes=False),
        scratch_types=[
            pltpu.VMEM((GCPS, 128), jnp.int32),
            pltpu.VMEM((GCPS, 128), jnp.int32),
            pltpu.VMEM_SHARED((NPAD, 144), jnp.float32),
            pltpu.VMEM((CHUNK, 144), jnp.float32),
            pltpu.VMEM((CHUNK, 16), jnp.float32),
            pltpu.SemaphoreType.DMA,
            pltpu.SemaphoreType.DMA,
        ],
    )
    def k(hgb, adcb, srcb, dstb, zb, onum,
          src_v, dst_v, acc, r0, ad0, m0, m1):
        c = lax.axis_index("c")
        s = lax.axis_index("s")
        w = s * NC + c
        pltpu.sync_copy(zb.at[pl.ds(s * RPT, RPT)], acc.at[pl.ds(s * RPT, RPT)])
        plsc.subcore_barrier()

        def compute_chunk(r, ad):
            def grp(g, c3):
                ridx = jax.lax.iota(jnp.int32, 16) + g * 16
                for hh in range(HEADS):
                    col = jnp.full((16,), hh, jnp.int32)
                    asv = plsc.load_gather(r, [ridx, col + 128])
                    adv = plsc.load_gather(ad, [ridx, col])
                    cv = plsc.load_gather(ad, [ridx, col + 4])
                    e = asv + adv
                    e = jnp.where(e >= 0.0, e, 0.2 * e)
                    exv = jnp.exp(e - cv)
                    plsc.store_scatter(r, [ridx, col + 128], exv)
                    for f in range(hh * DH, (hh + 1) * DH):
                        fcol = jnp.full((16,), f, jnp.int32)
                        v = plsc.load_gather(r, [ridx, fcol])
                        plsc.store_scatter(r, [ridx, fcol], v * exv)
                return c3
            lax.fori_loop(0, CHUNK // 16, grp, 0)

        def seg(sg, carry):
            base = w * CPW + sg * GCPS
            pltpu.sync_copy(srcb.at[pl.ds(base, GCPS)], src_v)
            pltpu.sync_copy(dstb.at[pl.ds(base, GCPS)], dst_v)

            def it(j, c2):
                d0 = pltpu.async_copy(hgb.at[src_v.at[j]], r0, m0)
                d1 = pltpu.async_copy(adcb.at[dst_v.at[j]], ad0, m1)
                d0.wait()
                d1.wait()
                compute_chunk(r0, ad0)
                pltpu.sync_copy(r0, acc.at[dst_v.at[j]], add=True)
                return c2
            lax.fori_loop(0, GCPS, it, 0)
            return carry
        lax.fori_loop(0, GSEG, seg, 0)

        plsc.subcore_barrier()
        pltpu.sync_copy(acc.at[pl.ds(s * RPT, RPT)],
                        onum.at[c, pl.ds(s * RPT, RPT)])

    return k(hgs, adc, src2d, dst2d, z144)


def _pool_sc(gat, batchp):
    @functools.partial(
        pl.kernel,
        out_type=(jax.ShapeDtypeStruct((NQ, GP, H), jnp.float32),
                  jax.ShapeDtypeStruct((NQ, GP, H), jnp.float32),
                  jax.ShapeDtypeStruct((NQ, GP, 16), jnp.float32)),
        mesh=_mesh,
        compiler_params=pltpu.CompilerParams(
            use_tc_tiling_on_sc=False, needs_layout_passes=False),
        scratch_types=[
            pltpu.VMEM((NPQ, 16), jnp.float32),
            pltpu.VMEM((NPQ + 16,), jnp.int32),
            pltpu.VMEM((GP, 16), jnp.float32),
            pltpu.VMEM((GP, 16), jnp.float32),
            pltpu.VMEM((GP, 16), jnp.float32),
        ],
    )
    def k(gatb, batb, osum, omax, ocnt, gv, bv, sacc, macc, cacc):
        c = lax.axis_index("c")
        s = lax.axis_index("s")
        w = s * NC + c
        q = w // 8
        fs = w % 8
        pltpu.sync_copy(
            gatb.at[pl.ds(q * NPQ, NPQ), pl.ds(fs * 16, 16)], gv)
        pltpu.sync_copy(batb.at[pl.ds(q * NPQ, NPQ)], bv.at[pl.ds(0, NPQ)])

        def init(i, carry):
            sacc[i, :] = jnp.zeros((16,), jnp.float32)
            cacc[i, :] = jnp.zeros((16,), jnp.float32)
            macc[i, :] = jnp.full((16,), -3.4e38, jnp.float32)
            return carry
        lax.fori_loop(0, GP, init, 0)

        def node(i, carry):
            g = bv[pl.ds(i, 16)][0]
            row = gv[i, :]
            sacc[g, :] = sacc[g, :] + row
            macc[g, :] = jnp.maximum(macc[g, :], row)
            cacc[g, :] = cacc[g, :] + 1.0
            return carry
        lax.fori_loop(0, NPQ, node, 0)

        pltpu.sync_copy(sacc, osum.at[q, :, pl.ds(fs * 16, 16)])
        pltpu.sync_copy(macc, omax.at[q, :, pl.ds(fs * 16, 16)])

        @pl.when(fs == 0)
        def _():
            pltpu.sync_copy(cacc, ocnt.at[q])

    return k(gat, batchp)


def _tc_first(x, w0, dinv_col):
    def body(x_ref, w_ref, di_ref, o_ref):
        o_ref[...] = jnp.dot(x_ref[...], w_ref[...],
                             preferred_element_type=jnp.float32) * di_ref[...]
    return pl.pallas_call(
        body, out_shape=jax.ShapeDtypeStruct((N, H), jnp.float32))(
            x, w0, dinv_col)


def _bn_relu(u, g, bb):
    m = jnp.mean(u, axis=0, keepdims=True)
    d = u - m
    v = jnp.mean(d * d, axis=0, keepdims=True)
    return jnp.maximum(d * jax.lax.rsqrt(v + 1e-5) * g + bb, 0.0)


def _tc_mid(P, hs, dinv_col, b, g, bb, wn):
    def body(p_ref, hs_ref, di_ref, b_ref, g_ref, bb_ref, w_ref, o_ref):
        u = (p_ref[0, :N, :] + p_ref[1, :N, :] + hs_ref[...]) * di_ref[...] \
            + b_ref[...]
        hr = _bn_relu(u, g_ref[...], bb_ref[...])
        o_ref[...] = jnp.dot(hr, w_ref[...],
                             preferred_element_type=jnp.float32) * di_ref[...]
    return pl.pallas_call(
        body, out_shape=jax.ShapeDtypeStruct((N, H), jnp.float32))(
            P, hs, dinv_col, b, g, bb, wn)


def _tc_gat_prep(P, hs, dinv_col, b, g, bb, wg, asrc, adst):
    def body(p_ref, hs_ref, di_ref, b_ref, g_ref, bb_ref, wg_ref,
             asrc_ref, adst_ref, hgs_out, adc_out):
        u = (p_ref[0, :N, :] + p_ref[1, :N, :] + hs_ref[...]) * di_ref[...] \
            + b_ref[...]
        h3 = _bn_relu(u, g_ref[...], bb_ref[...])
        hg = jnp.dot(h3, wg_ref[...], preferred_element_type=jnp.float32)
        cols_as = []
        cols_ad = []
        for hh in range(HEADS):
            blk = hg[:, hh * DH:(hh + 1) * DH]
            cols_as.append(jnp.sum(blk * asrc_ref[hh][None, :], axis=1,
                                   keepdims=True))
            cols_ad.append(jnp.sum(blk * adst_ref[hh][None, :], axis=1,
                                   keepdims=True))
        as_ = jnp.concatenate(cols_as, axis=1)
        ad_ = jnp.concatenate(cols_ad, axis=1)
        max_s = jnp.max(as_, axis=0, keepdims=True)
        cm = max_s + ad_
        cc = jnp.where(cm >= 0.0, cm, 0.2 * cm)
        hgs_out[0:N, :] = jnp.concatenate(
            [hg, as_, jnp.zeros((N, 12), jnp.float32)], axis=1)
        hgs_out[N:NP2, :] = jnp.zeros((NP2 - N, 144), jnp.float32)
        adc_out[0:N, :] = jnp.concatenate(
            [ad_, cc, jnp.zeros((N, 8), jnp.float32)], axis=1)
        adc_out[N:NP2, :] = jnp.zeros((NP2 - N, 16), jnp.float32)
    return pl.pallas_call(
        body,
        out_shape=(jax.ShapeDtypeStruct((NP2, 144), jnp.float32),
                   jax.ShapeDtypeStruct((NP2, 16), jnp.float32)))(
            P, hs, dinv_col, b, g, bb, wg, asrc, adst)


def _tc_gat_fin(onum, hgs, adc, bg):
    BR = NP2 // 4

    def body(on_ref, hgs_ref, adc_ref, bg_ref, o_ref):
        as_ = hgs_ref[:, 128:132]
        ad_ = adc_ref[:, 0:4]
        cc = adc_ref[:, 4:8]
        es = as_ + ad_
        es = jnp.where(es >= 0.0, es, 0.2 * es)
        exs = jnp.exp(es - cc)
        cols = []
        for hh in range(HEADS):
            sl = slice(hh * DH, (hh + 1) * DH)
            exh = exs[:, hh:hh + 1]
            den = (on_ref[0, :, 128 + hh:129 + hh]
                   + on_ref[1, :, 128 + hh:129 + hh] + exh)
            num = (on_ref[0, :, sl] + on_ref[1, :, sl]
                   + exh * hgs_ref[:, sl])
            cols.append(num / jnp.maximum(den, 1e-16)
                        + bg_ref[sl][None, :])
        o_ref[...] = jnp.concatenate(cols, axis=1)

    return pl.pallas_call(
        body,
        grid=(NP2 // BR,),
        in_specs=[
            pl.BlockSpec((NC, BR, 144), lambda i: (0, i, 0)),
            pl.BlockSpec((BR, 144), lambda i: (i, 0)),
            pl.BlockSpec((BR, 16), lambda i: (i, 0)),
            pl.BlockSpec((H,), lambda i: (0,)),
        ],
        out_specs=pl.BlockSpec((BR, H), lambda i: (i, 0)),
        out_shape=jax.ShapeDtypeStruct((NP2, H), jnp.float32))(
            onum, hgs, adc, bg)


def _tc_final(osum, omax, ocnt, w1, b1, w2, b2):
    def body(os_ref, om_ref, oc_ref, w1_ref, b1_ref, w2_ref, b2_ref,
             out_ref):
        sm = (os_ref[0, :G, :] + os_ref[1, :G, :]
              + os_ref[2, :G, :] + os_ref[3, :G, :])
        mx = jnp.maximum(jnp.maximum(om_ref[0, :G, :], om_ref[1, :G, :]),
                         jnp.maximum(om_ref[2, :G, :], om_ref[3, :G, :]))
        cnt = (oc_ref[0, :G, 0:1] + oc_ref[1, :G, 0:1]
               + oc_ref[2, :G, 0:1] + oc_ref[3, :G, 0:1])
        mean_pool = sm / jnp.maximum(cnt, 1.0)
        max_pool = jnp.where(cnt > 0.0, mx, 0.0)
        hh = jnp.concatenate([mean_pool, max_pool], axis=1)
        for t in range(T):
            z = jnp.maximum(
                jnp.dot(hh, w1_ref[t], preferred_element_type=jnp.float32)
                + b1_ref[t], 0.0)
            out_ref[:, t] = (jnp.dot(z, w2_ref[t],
                                     preferred_element_type=jnp.float32)
                             + b2_ref[t])[:, 0]
    return pl.pallas_call(
        body, out_shape=jax.ShapeDtypeStruct((G, T), jnp.float32))(
            osum, omax, ocnt, w1, b1, w2, b2)


def _heads_body(hh_ref, w1_ref, b1_ref, w2_ref, b2_ref, out_ref):
    hh = hh_ref[...]
    for t in range(T):
        z = jnp.maximum(
            jnp.dot(hh, w1_ref[t], preferred_element_type=jnp.float32)
            + b1_ref[t], 0.0)
        out_ref[:, t] = (jnp.dot(z, w2_ref[t],
                                 preferred_element_type=jnp.float32)
                         + b2_ref[t])[:, 0]


def _heads(hh, w1, b1, w2, b2):
    return pl.pallas_call(
        _heads_body,
        out_shape=jax.ShapeDtypeStruct((G, T), jnp.float32),
    )(hh, w1, b1, w2, b2)


def kernel(x, params, edge_index, batch):
    p = params
    src = edge_index[0].astype(jnp.int32)
    dst = edge_index[1].astype(jnp.int32)
    pad = EPAD - E
    src2d = jnp.concatenate(
        [src, jnp.zeros((pad,), jnp.int32)]).reshape(NBLK, 128)
    dst2d = jnp.concatenate(
        [dst, jnp.full((pad,), N, jnp.int32)]).reshape(NBLK, 128)
    z128 = jnp.zeros((NPAD, H), jnp.float32)

    dinv = _deg_dinv(dst2d)
    dinv_col = dinv[:N, None]

    h1s = _tc_first(x, p['W0'], dinv_col)
    P1 = _spmm(h1s, src2d, dst2d, z128)
    h2s = _tc_mid(P1, h1s, dinv_col, p['b0'], p['bn_g0'], p['bn_b0'], p['W1'])
    P2 = _spmm(h2s, src2d, dst2d, z128)
    h3s = _tc_mid(P2, h2s, dinv_col, p['b1'], p['bn_g1'], p['bn_b1'], p['W2'])
    P3 = _spmm(h3s, src2d, dst2d, z128)
    hgs, adc = _tc_gat_prep(P3, h3s, dinv_col, p['b2'], p['bn_g2'],
                            p['bn_b2'], p['Wg'], p['att_src'],
                            p['att_dst'])
    z144 = jnp.zeros((NPAD, 144), jnp.float32)
    onum = _gat_sc(hgs, adc, src2d, dst2d, z144)
    gat = _tc_gat_fin(onum, hgs, adc, p['bg'])
    batchp = jnp.concatenate(
        [batch.astype(jnp.int32), jnp.full((NP2 - N,), G, jnp.int32)])
    osum, omax, ocnt = _pool_sc(gat, batchp)
    w1 = jnp.stack([p['h%d_W1' % t] for t in range(T)])
    b1 = jnp.stack([p['h%d_b1' % t] for t in range(T)])
    w2 = jnp.stack([p['h%d_W2' % t] for t in range(T)])
    b2 = jnp.stack([p['h%d_b2' % t] for t in range(T)])
    return _tc_final(osum, omax, ocnt, w1, b1, w2, b2)


# speedup vs baseline: 18.4911x; 1.0664x over previous
import functools
import jax
import jax.numpy as jnp
from jax import lax
from jax.experimental import pallas as pl
from jax.experimental.pallas import tpu as pltpu
from jax.experimental.pallas import tpu_sc as plsc

N = 10000
E = 640000
ATOM = 128
H = 128
HEADS = 4
DH = 32
G = 256
T = 5

NC = 2
NS = 16
NW = NC * NS
CHUNK = 128
CPW = 160
EPW = CHUNK * CPW
EPAD = EPW * NW
NBLK = EPAD // CHUNK
NPAD = 10240
RPT = NPAD // NS
DPB = NBLK // NS
DNW = NPAD // NW
SEG = 4
CPS = CPW // SEG
GSEG = 8
GCPS = CPW // GSEG
C64 = 64
NB64 = EPAD // C64
CPW64 = NB64 // NW
SEG64 = 8
CPS64 = CPW64 // SEG64
NP2 = 10016
NQ = 4
NPQ = NP2 // NQ
GP = 272

_mesh = plsc.VectorSubcoreMesh(
    core_axis_name="c", subcore_axis_name="s", num_cores=NC, num_subcores=NS)


def _vslices(n):
    return range(n // 16)


def _deg_dinv(dst2d):
    @functools.partial(
        pl.kernel,
        out_type=jax.ShapeDtypeStruct((NPAD,), jnp.float32),
        mesh=_mesh,
        compiler_params=pltpu.CompilerParams(use_tc_tiling_on_sc=False, needs_layout_passes=False),
        scratch_types=[
            pltpu.VMEM((DPB, 128), jnp.int32),
            pltpu.VMEM((NPAD,), jnp.float32),
            pltpu.VMEM((DNW,), jnp.float32),
            pltpu.VMEM((DNW,), jnp.float32),
            pltpu.VMEM_SHARED((NS, NPAD), jnp.float32),
        ],
    )
    def k(dst_hbm, out_hbm, idx_v, pdeg, buf, tmp, sdeg):
        c = lax.axis_index("c")
        s = lax.axis_index("s")
        w = s * NC + c
        pltpu.sync_copy(dst_hbm.at[pl.ds(s * DPB, DPB)], idx_v)

        def z(i, carry):
            pdeg[pl.ds(i * 16, 16)] = jnp.zeros((16,), jnp.float32)
            return carry
        lax.fori_loop(0, NPAD // 16, z, 0)

        ones = jnp.ones((16,), jnp.float32)

        def cnt(j, carry):
            for g in range(8):
                idx = idx_v[j, pl.ds(g * 16, 16)]
                plsc.addupdate_scatter(pdeg, [idx], ones)
            return carry
        lax.fori_loop(0, DPB, cnt, 0)

        pltpu.sync_copy(pdeg, sdeg.at[s])
        plsc.subcore_barrier()

        base = w * DNW

        def zz(i, carry):
            buf[pl.ds(i * 16, 16)] = jnp.zeros((16,), jnp.float32)
            return carry
        lax.fori_loop(0, DNW // 16, zz, 0)

        def mrg(t, carry):
            pltpu.sync_copy(sdeg.at[t, pl.ds(base, DNW)], tmp)

            def add(i, c2):
                sl = pl.ds(i * 16, 16)
                buf[sl] = buf[sl] + tmp[sl]
                return c2
            lax.fori_loop(0, DNW // 16, add, 0)
            return carry
        lax.fori_loop(0, NS, mrg, 0)

        def rs(i, carry):
            sl = pl.ds(i * 16, 16)
            d = buf[sl] + 1.0
            ii = plsc.bitcast(d, jnp.int32)
            ii = 0x5F3759DF - jax.lax.shift_right_logical(ii, 1)
            y = plsc.bitcast(ii, jnp.float32)
            for _ in range(3):
                y = y * (1.5 - 0.5 * d * y * y)
            buf[sl] = y
            return carry
        lax.fori_loop(0, DNW // 16, rs, 0)

        pltpu.sync_copy(buf, out_hbm.at[pl.ds(base, DNW)])

    return k(dst2d)


def _spmm(table, src2d, dst2d, z128):
    @functools.partial(
        pl.kernel,
        out_type=jax.ShapeDtypeStruct((NC, NPAD, H), jnp.float32),
        mesh=_mesh,
        compiler_params=pltpu.CompilerParams(use_tc_tiling_on_sc=False, needs_layout_passes=False),
        scratch_types=[
            pltpu.VMEM((CPS64, C64), jnp.int32),
            pltpu.VMEM((CPS64, C64), jnp.int32),
            pltpu.VMEM_SHARED((NPAD, H), jnp.float32),
            pltpu.VMEM((C64, H), jnp.float32),
            pltpu.VMEM((C64, H), jnp.float32),
            pltpu.VMEM((C64, H), jnp.float32),
            pltpu.VMEM((C64, H), jnp.float32),
            pltpu.SemaphoreType.DMA,
            pltpu.SemaphoreType.DMA,
            pltpu.SemaphoreType.DMA,
            pltpu.SemaphoreType.DMA,
            pltpu.SemaphoreType.DMA,
            pltpu.SemaphoreType.DMA,
            pltpu.SemaphoreType.DMA,
            pltpu.SemaphoreType.DMA,
        ],
    )
    def k(tab, srcb, dstb, zb, out, src_v, dst_v, acc, r0, r1, r2, r3,
          s0, s1, s2, s3, t0, t1, t2, t3):
        c = lax.axis_index("c")
        s = lax.axis_index("s")
        w = s * NC + c
        pltpu.sync_copy(zb.at[pl.ds(s * RPT, RPT)], acc.at[pl.ds(s * RPT, RPT)])
        plsc.subcore_barrier()
        rows = [r0, r1, r2, r3]
        gsem = [s0, s1, s2, s3]
        tsem = [t0, t1, t2, t3]

        def seg(sg, carry):
            base = w * CPW64 + sg * CPS64
            pltpu.sync_copy(srcb.at[pl.ds(base, CPS64)], src_v)
            pltpu.sync_copy(dstb.at[pl.ds(base, CPS64)], dst_v)

            def it(jq, c2):
                j0 = jq * 4
                gd = [pltpu.async_copy(tab.at[src_v.at[j0 + b]], rows[b],
                                       gsem[b]) for b in range(4)]
                sd = []
                for b in range(4):
                    gd[b].wait()
                    sd.append(pltpu.async_copy(
                        rows[b], acc.at[dst_v.at[j0 + b]], tsem[b],
                        add=True))
                for b in range(4):
                    sd[b].wait()
                return c2
            lax.fori_loop(0, CPS64 // 4, it, 0)
            return carry
        lax.fori_loop(0, SEG64, seg, 0)

        plsc.subcore_barrier()
        pltpu.sync_copy(acc.at[pl.ds(s * RPT, RPT)],
                        out.at[c, pl.ds(s * RPT, RPT)])

    return k(table, src2d, dst2d, z128)


def _gat_sc(hgs, adc, src64, dst64, z144):
    @functools.partial(
        pl.kernel,
        out_type=jax.ShapeDtypeStruct((NC, NPAD, 144), jnp.float32),
        mesh=_mesh,
        compiler_params=pltpu.CompilerParams(
            use_tc_tiling_on_sc=False, needs_layout_passes=False),
        scratch_types=[
            pltpu.VMEM((CPS64, C64), jnp.int32),
            pltpu.VMEM((CPS64, C64), jnp.int32),
            pltpu.VMEM_SHARED((NPAD, 144), jnp.float32),
            pltpu.VMEM((C64, 144), jnp.float32),
            pltpu.VMEM((C64, 144), jnp.float32),
            pltpu.VMEM((C64, 144), jnp.float32),
            pltpu.VMEM((C64, 16), jnp.float32),
            pltpu.VMEM((C64, 16), jnp.float32),
            pltpu.VMEM((C64, 16), jnp.float32),
            pltpu.SemaphoreType.DMA,
            pltpu.SemaphoreType.DMA,
            pltpu.SemaphoreType.DMA,
            pltpu.SemaphoreType.DMA,
            pltpu.SemaphoreType.DMA,
            pltpu.SemaphoreType.DMA,
            pltpu.SemaphoreType.DMA,
            pltpu.SemaphoreType.DMA,
            pltpu.SemaphoreType.DMA,
        ],
    )
    def k(hgb, adcb, srcb, dstb, zb, onum,
          src_v, dst_v, acc, r0, r1, r2, ad0, ad1, ad2,
          g0, g1, g2, a0, a1, a2, t0, t1, t2):
        c = lax.axis_index("c")
        s = lax.axis_index("s")
        w = s * NC + c
        pltpu.sync_copy(zb.at[pl.ds(s * RPT, RPT)], acc.at[pl.ds(s * RPT, RPT)])
        plsc.subcore_barrier()
        rows = [r0, r1, r2]
        ads = [ad0, ad1, ad2]
        gsem = [g0, g1, g2]
        asem = [a0, a1, a2]
        tsem = [t0, t1, t2]

        def compute_chunk(r, ad):
            def grp(g, c3):
                ridx = jax.lax.iota(jnp.int32, 16) + g * 16
                for hh in range(HEADS):
                    col = jnp.full((16,), hh, jnp.int32)
                    asv = plsc.load_gather(r, [ridx, col + 128])
                    adv = plsc.load_gather(ad, [ridx, col])
                    cv = plsc.load_gather(ad, [ridx, col + 4])
                    e = asv + adv
                    e = jnp.where(e >= 0.0, e, 0.2 * e)
                    exv = jnp.exp(e - cv)
                    plsc.store_scatter(r, [ridx, col + 128], exv)
                    for f in range(hh * DH, (hh + 1) * DH):
                        fcol = jnp.full((16,), f, jnp.int32)
                        v = plsc.load_gather(r, [ridx, fcol])
                        plsc.store_scatter(r, [ridx, fcol], v * exv)
                return c3
            lax.fori_loop(0, C64 // 16, grp, 0)

        def seg(sg, carry):
            base = w * CPW64 + sg * CPS64
            pltpu.sync_copy(srcb.at[pl.ds(base, CPS64)], src_v)
            pltpu.sync_copy(dstb.at[pl.ds(base, CPS64)], dst_v)

            def it(jq, c2):
                j0 = jq * 3
                gd = []
                for b in range(3):
                    gd.append((
                        pltpu.async_copy(hgb.at[src_v.at[j0 + b]], rows[b],
                                         gsem[b]),
                        pltpu.async_copy(adcb.at[dst_v.at[j0 + b]], ads[b],
                                         asem[b])))
                sd = []
                for b in range(3):
                    gd[b][0].wait()
                    gd[b][1].wait()
                    compute_chunk(rows[b], ads[b])
                    sd.append(pltpu.async_copy(
                        rows[b], acc.at[dst_v.at[j0 + b]], tsem[b],
                        add=True))
                for b in range(3):
                    sd[b].wait()
                return c2
            lax.fori_loop(0, CPS64 // 3, it, 0)

            j = CPS64 - 1
            d0 = pltpu.async_copy(hgb.at[src_v.at[j]], r0, g0)
            d1 = pltpu.async_copy(adcb.at[dst_v.at[j]], ad0, a0)
            d0.wait()
            d1.wait()
            compute_chunk(r0, ad0)
            pltpu.sync_copy(r0, acc.at[dst_v.at[j]], add=True)
            return carry
        lax.fori_loop(0, GSEG, seg, 0)

        plsc.subcore_barrier()
        pltpu.sync_copy(acc.at[pl.ds(s * RPT, RPT)],
                        onum.at[c, pl.ds(s * RPT, RPT)])

    return k(hgs, adc, src64, dst64, z144)


def _pool_sc(gat, batchp):
    @functools.partial(
        pl.kernel,
        out_type=(jax.ShapeDtypeStruct((NQ, GP, H), jnp.float32),
                  jax.ShapeDtypeStruct((NQ, GP, H), jnp.float32),
                  jax.ShapeDtypeStruct((NQ, GP, 16), jnp.float32)),
        mesh=_mesh,
        compiler_params=pltpu.CompilerParams(
            use_tc_tiling_on_sc=False, needs_layout_passes=False),
        scratch_types=[
            pltpu.VMEM((NPQ, 16), jnp.float32),
            pltpu.VMEM((NPQ + 16,), jnp.int32),
            pltpu.VMEM((GP, 16), jnp.float32),
            pltpu.VMEM((GP, 16), jnp.float32),
            pltpu.VMEM((GP, 16), jnp.float32),
        ],
    )
    def k(gatb, batb, osum, omax, ocnt, gv, bv, sacc, macc, cacc):
        c = lax.axis_index("c")
        s = lax.axis_index("s")
        w = s * NC + c
        q = w // 8
        fs = w % 8
        pltpu.sync_copy(
            gatb.at[pl.ds(q * NPQ, NPQ), pl.ds(fs * 16, 16)], gv)
        pltpu.sync_copy(batb.at[pl.ds(q * NPQ, NPQ)], bv.at[pl.ds(0, NPQ)])

        def init(i, carry):
            sacc[i, :] = jnp.zeros((16,), jnp.float32)
            cacc[i, :] = jnp.zeros((16,), jnp.float32)
            macc[i, :] = jnp.full((16,), -3.4e38, jnp.float32)
            return carry
        lax.fori_loop(0, GP, init, 0)

        def node(i, carry):
            g = bv[pl.ds(i, 16)][0]
            row = gv[i, :]
            sacc[g, :] = sacc[g, :] + row
            macc[g, :] = jnp.maximum(macc[g, :], row)
            cacc[g, :] = cacc[g, :] + 1.0
            return carry
        lax.fori_loop(0, NPQ, node, 0)

        pltpu.sync_copy(sacc, osum.at[q, :, pl.ds(fs * 16, 16)])
        pltpu.sync_copy(macc, omax.at[q, :, pl.ds(fs * 16, 16)])

        @pl.when(fs == 0)
        def _():
            pltpu.sync_copy(cacc, ocnt.at[q])

    return k(gat, batchp)


def _tc_first(x, w0, dinv_col):
    def body(x_ref, w_ref, di_ref, o_ref):
        o_ref[...] = jnp.dot(x_ref[...], w_ref[...],
                             preferred_element_type=jnp.float32) * di_ref[...]
    return pl.pallas_call(
        body, out_shape=jax.ShapeDtypeStruct((N, H), jnp.float32))(
            x, w0, dinv_col)


def _bn_relu(u, g, bb):
    m = jnp.mean(u, axis=0, keepdims=True)
    d = u - m
    v = jnp.mean(d * d, axis=0, keepdims=True)
    return jnp.maximum(d * jax.lax.rsqrt(v + 1e-5) * g + bb, 0.0)


def _tc_mid(P, hs, dinv_col, b, g, bb, wn):
    def body(p_ref, hs_ref, di_ref, b_ref, g_ref, bb_ref, w_ref, o_ref):
        u = (p_ref[0, :N, :] + p_ref[1, :N, :] + hs_ref[...]) * di_ref[...] \
            + b_ref[...]
        hr = _bn_relu(u, g_ref[...], bb_ref[...])
        o_ref[...] = jnp.dot(hr, w_ref[...],
                             preferred_element_type=jnp.float32) * di_ref[...]
    return pl.pallas_call(
        body, out_shape=jax.ShapeDtypeStruct((N, H), jnp.float32))(
            P, hs, dinv_col, b, g, bb, wn)


def _tc_gat_prep(P, hs, dinv_col, b, g, bb, wg, asrc, adst):
    def body(p_ref, hs_ref, di_ref, b_ref, g_ref, bb_ref, wg_ref,
             asrc_ref, adst_ref, hgs_out, adc_out):
        u = (p_ref[0, :N, :] + p_ref[1, :N, :] + hs_ref[...]) * di_ref[...] \
            + b_ref[...]
        h3 = _bn_relu(u, g_ref[...], bb_ref[...])
        hg = jnp.dot(h3, wg_ref[...], preferred_element_type=jnp.float32)
        cols_as = []
        cols_ad = []
        for hh in range(HEADS):
            blk = hg[:, hh * DH:(hh + 1) * DH]
            cols_as.append(jnp.sum(blk * asrc_ref[hh][None, :], axis=1,
                                   keepdims=True))
            cols_ad.append(jnp.sum(blk * adst_ref[hh][None, :], axis=1,
                                   keepdims=True))
        as_ = jnp.concatenate(cols_as, axis=1)
        ad_ = jnp.concatenate(cols_ad, axis=1)
        max_s = jnp.max(as_, axis=0, keepdims=True)
        cm = max_s + ad_
        cc = jnp.where(cm >= 0.0, cm, 0.2 * cm)
        hgs_out[0:N, :] = jnp.concatenate(
            [hg, as_, jnp.zeros((N, 12), jnp.float32)], axis=1)
        hgs_out[N:NP2, :] = jnp.zeros((NP2 - N, 144), jnp.float32)
        adc_out[0:N, :] = jnp.concatenate(
            [ad_, cc, jnp.zeros((N, 8), jnp.float32)], axis=1)
        adc_out[N:NP2, :] = jnp.zeros((NP2 - N, 16), jnp.float32)
    return pl.pallas_call(
        body,
        out_shape=(jax.ShapeDtypeStruct((NP2, 144), jnp.float32),
                   jax.ShapeDtypeStruct((NP2, 16), jnp.float32)))(
            P, hs, dinv_col, b, g, bb, wg, asrc, adst)


def _tc_gat_fin(onum, hgs, adc, bg):
    BR = NP2 // 4

    def body(on_ref, hgs_ref, adc_ref, bg_ref, o_ref):
        as_ = hgs_ref[:, 128:132]
        ad_ = adc_ref[:, 0:4]
        cc = adc_ref[:, 4:8]
        es = as_ + ad_
        es = jnp.where(es >= 0.0, es, 0.2 * es)
        exs = jnp.exp(es - cc)
        cols = []
        for hh in range(HEADS):
            sl = slice(hh * DH, (hh + 1) * DH)
            exh = exs[:, hh:hh + 1]
            den = (on_ref[0, :, 128 + hh:129 + hh]
                   + on_ref[1, :, 128 + hh:129 + hh] + exh)
            num = (on_ref[0, :, sl] + on_ref[1, :, sl]
                   + exh * hgs_ref[:, sl])
            cols.append(num / jnp.maximum(den, 1e-16)
                        + bg_ref[sl][None, :])
        o_ref[...] = jnp.concatenate(cols, axis=1)

    return pl.pallas_call(
        body,
        grid=(NP2 // BR,),
        in_specs=[
            pl.BlockSpec((NC, BR, 144), lambda i: (0, i, 0)),
            pl.BlockSpec((BR, 144), lambda i: (i, 0)),
            pl.BlockSpec((BR, 16), lambda i: (i, 0)),
            pl.BlockSpec((H,), lambda i: (0,)),
        ],
        out_specs=pl.BlockSpec((BR, H), lambda i: (i, 0)),
        out_shape=jax.ShapeDtypeStruct((NP2, H), jnp.float32))(
            onum, hgs, adc, bg)


def _tc_final(osum, omax, ocnt, w1, b1, w2, b2):
    def body(os_ref, om_ref, oc_ref, w1_ref, b1_ref, w2_ref, b2_ref,
             out_ref):
        sm = (os_ref[0, :G, :] + os_ref[1, :G, :]
              + os_ref[2, :G, :] + os_ref[3, :G, :])
        mx = jnp.maximum(jnp.maximum(om_ref[0, :G, :], om_ref[1, :G, :]),
                         jnp.maximum(om_ref[2, :G, :], om_ref[3, :G, :]))
        cnt = (oc_ref[0, :G, 0:1] + oc_ref[1, :G, 0:1]
               + oc_ref[2, :G, 0:1] + oc_ref[3, :G, 0:1])
        mean_pool = sm / jnp.maximum(cnt, 1.0)
        max_pool = jnp.where(cnt > 0.0, mx, 0.0)
        hh = jnp.concatenate([mean_pool, max_pool], axis=1)
        for t in range(T):
            z = jnp.maximum(
                jnp.dot(hh, w1_ref[t], preferred_element_type=jnp.float32)
                + b1_ref[t], 0.0)
            out_ref[:, t] = (jnp.dot(z, w2_ref[t],
                                     preferred_element_type=jnp.float32)
                             + b2_ref[t])[:, 0]
    return pl.pallas_call(
        body, out_shape=jax.ShapeDtypeStruct((G, T), jnp.float32))(
            osum, omax, ocnt, w1, b1, w2, b2)


def _heads_body(hh_ref, w1_ref, b1_ref, w2_ref, b2_ref, out_ref):
    hh = hh_ref[...]
    for t in range(T):
        z = jnp.maximum(
            jnp.dot(hh, w1_ref[t], preferred_element_type=jnp.float32)
            + b1_ref[t], 0.0)
        out_ref[:, t] = (jnp.dot(z, w2_ref[t],
                                 preferred_element_type=jnp.float32)
                         + b2_ref[t])[:, 0]


def _heads(hh, w1, b1, w2, b2):
    return pl.pallas_call(
        _heads_body,
        out_shape=jax.ShapeDtypeStruct((G, T), jnp.float32),
    )(hh, w1, b1, w2, b2)


def kernel(x, params, edge_index, batch):
    p = params
    src = edge_index[0].astype(jnp.int32)
    dst = edge_index[1].astype(jnp.int32)
    pad = EPAD - E
    src_p = jnp.concatenate([src, jnp.zeros((pad,), jnp.int32)])
    dst_p = jnp.concatenate([dst, jnp.full((pad,), N, jnp.int32)])
    dst2d = dst_p.reshape(NBLK, 128)
    src64 = src_p.reshape(NB64, C64)
    dst64 = dst_p.reshape(NB64, C64)
    z128 = jnp.zeros((NPAD, H), jnp.float32)

    dinv = _deg_dinv(dst2d)
    dinv_col = dinv[:N, None]

    h1s = _tc_first(x, p['W0'], dinv_col)
    P1 = _spmm(h1s, src64, dst64, z128)
    h2s = _tc_mid(P1, h1s, dinv_col, p['b0'], p['bn_g0'], p['bn_b0'], p['W1'])
    P2 = _spmm(h2s, src64, dst64, z128)
    h3s = _tc_mid(P2, h2s, dinv_col, p['b1'], p['bn_g1'], p['bn_b1'], p['W2'])
    P3 = _spmm(h3s, src64, dst64, z128)
    hgs, adc = _tc_gat_prep(P3, h3s, dinv_col, p['b2'], p['bn_g2'],
                            p['bn_b2'], p['Wg'], p['att_src'],
                            p['att_dst'])
    z144 = jnp.zeros((NPAD, 144), jnp.float32)
    onum = _gat_sc(hgs, adc, src64, dst64, z144)
    gat = _tc_gat_fin(onum, hgs, adc, p['bg'])
    batchp = jnp.concatenate(
        [batch.astype(jnp.int32), jnp.full((NP2 - N,), G, jnp.int32)])
    osum, omax, ocnt = _pool_sc(gat, batchp)
    w1 = jnp.stack([p['h%d_W1' % t] for t in range(T)])
    b1 = jnp.stack([p['h%d_b1' % t] for t in range(T)])
    w2 = jnp.stack([p['h%d_W2' % t] for t in range(T)])
    b2 = jnp.stack([p['h%d_b2' % t] for t in range(T)])
    return _tc_final(osum, omax, ocnt, w1, b1, w2, b2)
